# baseline (speedup 1.0000x reference)
import functools

import jax
import jax.numpy as jnp
from jax import lax
from jax.experimental import pallas as pl
from jax.experimental.pallas import tpu as pltpu
from jax.experimental.pallas import tpu_sc as plsc

N = 10000
E = 320000
G = 64
NP = 10240
RPT = NP // 16
EP = 327680
C = 128
K = 80
TRASH = 10016
W72 = 72
F32 = jnp.float32

_sc_mesh = functools.partial(
    plsc.VectorSubcoreMesh, core_axis_name="c", subcore_axis_name="s")


def _wid():
    return lax.axis_index("s") * 2 + lax.axis_index("c")


def _make_deg():
    def body(dst3, ones, zeros, out, idx_d, ones_v, acc, semA, semB):
        c = lax.axis_index("c")
        s = lax.axis_index("s")
        w = _wid()
        pltpu.sync_copy(ones, ones_v)
        pltpu.sync_copy(zeros.at[pl.ds(s * RPT, RPT)], acc.at[pl.ds(s * RPT, RPT)])
        pltpu.sync_copy(dst3.at[w], idx_d)
        plsc.subcore_barrier()

        def scat(j, sem):
            pltpu.async_copy(ones_v, acc.at[idx_d.at[j]], sem, add=True)

        def wait_scat(j, sem):
            pltpu.make_async_copy(ones_v, acc.at[idx_d.at[j]], sem).wait()

        scat(0, semA)

        def pair(k, carry):
            j0 = 2 * k + 1
            scat(j0, semB)
            wait_scat(j0 - 1, semA)
            scat(j0 + 1, semA)
            wait_scat(j0, semB)
            return carry

        lax.fori_loop(0, (K - 2) // 2, pair, 0)
        scat(K - 1, semB)
        wait_scat(K - 2, semA)
        wait_scat(K - 1, semB)
        plsc.subcore_barrier()
        pltpu.sync_copy(acc.at[pl.ds(s * RPT, RPT)],
                        out.at[c].at[pl.ds(s * RPT, RPT)])

    return pl.kernel(
        body,
        out_type=jax.ShapeDtypeStruct((2, NP, 16), F32),
        mesh=_sc_mesh(),
        compiler_params=pltpu.CompilerParams(
            use_tc_tiling_on_sc=False, needs_layout_passes=False),
        scratch_types=[
            pltpu.VMEM((K, C), jnp.int32),
            pltpu.VMEM((C, 16), F32),
            pltpu.VMEM_SHARED((NP, 16), F32),
            pltpu.SemaphoreType.DMA,
            pltpu.SemaphoreType.DMA,
        ],
    )


def _make_gcn():
    def body(hp, src3, dst3, zeros, out, idx_s, idx_d, rows0, rows1, rows2,
             acc, sem0, sem1, sem2, ssc0, ssc1, ssc2):
        c = lax.axis_index("c")
        s = lax.axis_index("s")
        w = _wid()
        pltpu.sync_copy(zeros.at[pl.ds(s * RPT, RPT)], acc.at[pl.ds(s * RPT, RPT)])
        pltpu.sync_copy(src3.at[w], idx_s.at[pl.ds(0, K)])
        pltpu.sync_copy(dst3.at[w], idx_d)
        for k in range(C // 16):
            idx_s[K, pl.ds(k * 16, 16)] = jnp.zeros((16,), jnp.int32)
        plsc.subcore_barrier()

        def gather(j, buf, sem):
            return pltpu.async_copy(hp.at[idx_s.at[j]], buf, sem)

        def wait_gather(j, buf, sem):
            pltpu.make_async_copy(hp.at[idx_s.at[j]], buf, sem).wait()

        def scat(j, buf, sem):
            pltpu.async_copy(buf, acc.at[idx_d.at[j]], sem, add=True)

        def wait_scat(j, buf, sem):
            pltpu.make_async_copy(buf, acc.at[idx_d.at[j]], sem).wait()

        bufs = (rows0, rows1, rows2)
        gsem = (ssc0, ssc1, ssc2)
        ssem = (sem0, sem1, sem2)

        gather(0, rows0, ssc0)
        wait_gather(0, rows0, ssc0)
        scat(0, rows0, sem0)
        gather(1, rows1, ssc1)
        wait_gather(1, rows1, ssc1)
        scat(1, rows1, sem1)
        gather(2, rows2, ssc2)

        def round_(k, carry):
            j0 = 3 * k + 2
            for i in range(3):
                j = j0 + i
                b = (2 + i) % 3
                nb = (b + 1) % 3
                wait_gather(j, bufs[b], gsem[b])
                scat(j, bufs[b], ssem[b])
                wait_scat(j - 2, bufs[nb], ssem[nb])
                gather(j + 1, bufs[nb], gsem[nb])
            return carry

        lax.fori_loop(0, (K - 2) // 3, round_, 0)
        wait_gather(K, bufs[2], gsem[2])
        wait_scat(K - 2, bufs[0], ssem[0])
        wait_scat(K - 1, bufs[1], ssem[1])

        plsc.subcore_barrier()
        pltpu.sync_copy(acc.at[pl.ds(s * RPT, RPT)],
                        out.at[c].at[pl.ds(s * RPT, RPT)])

    return pl.kernel(
        body,
        out_type=jax.ShapeDtypeStruct((2, NP, 64), F32),
        mesh=_sc_mesh(),
        compiler_params=pltpu.CompilerParams(
            use_tc_tiling_on_sc=False, needs_layout_passes=False),
        scratch_types=[
            pltpu.VMEM((K + 1, C), jnp.int32),
            pltpu.VMEM((K, C), jnp.int32),
            pltpu.VMEM((C, 64), F32),
            pltpu.VMEM((C, 64), F32),
            pltpu.VMEM((C, 64), F32),
            pltpu.VMEM_SHARED((NP, 64), F32),
            pltpu.SemaphoreType.DMA,
            pltpu.SemaphoreType.DMA,
            pltpu.SemaphoreType.DMA,
            pltpu.SemaphoreType.DMA,
            pltpu.SemaphoreType.DMA,
            pltpu.SemaphoreType.DMA,
        ],
    )


def _make_agnn():
    def compute_chunk(rows_s, rows_d, bvec):
        c65 = jnp.full((16,), 65, jnp.int32)

        def grp(gi, carry):
            cvec = lax.iota(jnp.int32, 16) + gi * 16

            def fq(q, a):
                for k in range(4):
                    fv = lax.broadcast(q * 4 + k, (16,))
                    vs = plsc.load_gather(rows_s, [cvec, fv])
                    vd = plsc.load_gather(rows_d, [cvec, fv])
                    a = a + vs * vd
                return a

            dot = lax.fori_loop(0, 16, fq, jnp.zeros((16,), F32))
            ns = plsc.load_gather(rows_s, [cvec, c65])
            nd = plsc.load_gather(rows_d, [cvec, c65])
            ex = jnp.exp(bvec * dot * ns * nd)

            def fscale(f, carry2):
                fv = lax.broadcast(f, (16,))
                col = plsc.load_gather(rows_s, [cvec, fv])
                plsc.store_scatter(rows_s, [cvec, fv], col * ex)
                return carry2

            lax.fori_loop(0, 65, fscale, 0)
            return carry

        lax.fori_loop(0, C // 16, grp, 0)

    def body(xt, src3, dst3, beta16, zeros, out, idx_s, idx_d,
             rs0, rs1, rd0, rd1, beta_v, acc, ss0, ss1, sd0, sd1, sc0, sc1):
        c = lax.axis_index("c")
        s = lax.axis_index("s")
        w = _wid()
        pltpu.sync_copy(beta16, beta_v)
        pltpu.sync_copy(zeros.at[pl.ds(s * RPT, RPT)], acc.at[pl.ds(s * RPT, RPT)])
        pltpu.sync_copy(src3.at[w], idx_s.at[pl.ds(0, K)])
        pltpu.sync_copy(dst3.at[w], idx_d.at[pl.ds(0, K)])
        for k in range(C // 16):
            idx_s[K, pl.ds(k * 16, 16)] = jnp.zeros((16,), jnp.int32)
            idx_d[K, pl.ds(k * 16, 16)] = jnp.zeros((16,), jnp.int32)
        plsc.subcore_barrier()
        bvec = beta_v[...]

        def gathers(j, bs, bd, sems, semd):
            pltpu.async_copy(xt.at[idx_s.at[j]], bs, sems)
            pltpu.async_copy(xt.at[idx_d.at[j]], bd, semd)

        def waits(j, bs, bd, sems, semd):
            pltpu.make_async_copy(xt.at[idx_s.at[j]], bs, sems).wait()
            pltpu.make_async_copy(xt.at[idx_d.at[j]], bd, semd).wait()

        def scat(j, buf, sem):
            pltpu.async_copy(buf, acc.at[idx_d.at[j]], sem, add=True)

        def wait_scat(j, buf, sem):
            pltpu.make_async_copy(buf, acc.at[idx_d.at[j]], sem).wait()

        gathers(0, rs0, rd0, ss0, sd0)
        waits(0, rs0, rd0, ss0, sd0)
        gathers(1, rs1, rd1, ss1, sd1)
        compute_chunk(rs0, rd0, bvec)
        scat(0, rs0, sc0)

        def pair(k, carry):
            j0 = 2 * k + 1
            waits(j0, rs1, rd1, ss1, sd1)
            wait_scat(j0 - 1, rs0, sc0)
            gathers(j0 + 1, rs0, rd0, ss0, sd0)
            compute_chunk(rs1, rd1, bvec)
            scat(j0, rs1, sc1)
            waits(j0 + 1, rs0, rd0, ss0, sd0)
            wait_scat(j0, rs1, sc1)
            gathers(j0 + 2, rs1, rd1, ss1, sd1)
            compute_chunk(rs0, rd0, bvec)
            scat(j0 + 1, rs0, sc0)
            return carry

        lax.fori_loop(0, (K - 2) // 2, pair, 0)
        waits(K - 1, rs1, rd1, ss1, sd1)
        wait_scat(K - 2, rs0, sc0)
        compute_chunk(rs1, rd1, bvec)
        scat(K - 1, rs1, sc1)
        wait_scat(K - 1, rs1, sc1)

        plsc.subcore_barrier()
        pltpu.sync_copy(acc.at[pl.ds(s * RPT, RPT)],
                        out.at[c].at[pl.ds(s * RPT, RPT)])

    return pl.kernel(
        body,
        out_type=jax.ShapeDtypeStruct((2, NP, W72), F32),
        mesh=_sc_mesh(),
        compiler_params=pltpu.CompilerParams(
            use_tc_tiling_on_sc=False, needs_layout_passes=False),
        scratch_types=[
            pltpu.VMEM((K + 1, C), jnp.int32),
            pltpu.VMEM((K + 1, C), jnp.int32),
            pltpu.VMEM((C, W72), F32),
            pltpu.VMEM((C, W72), F32),
            pltpu.VMEM((C, W72), F32),
            pltpu.VMEM((C, W72), F32),
            pltpu.VMEM((16,), F32),
            pltpu.VMEM_SHARED((NP, W72), F32),
            pltpu.SemaphoreType.DMA,
            pltpu.SemaphoreType.DMA,
            pltpu.SemaphoreType.DMA,
            pltpu.SemaphoreType.DMA,
            pltpu.SemaphoreType.DMA,
            pltpu.SemaphoreType.DMA,
        ],
    )


_sc_deg = _make_deg()
_sc_gcn = _make_gcn()
_sc_agnn = _make_agnn()


def _t0_body(degp, xp, w1, h_out, dinv_out):
    deg = degp[0, :, 0:1] + degp[1, :, 0:1] + 1.0
    dinv = lax.rsqrt(deg)
    dinv_out[...] = dinv
    h_out[...] = jnp.dot(xp[...], w1[...], preferred_element_type=F32) * dinv


def _t0(degp, xp, w1):
    return pl.pallas_call(
        _t0_body,
        out_shape=[jax.ShapeDtypeStruct((NP, 64), F32),
                   jax.ShapeDtypeStruct((NP, 1), F32)],
    )(degp, xp, w1)


def _tmid_body(accp, hp, dinv, b, wn, h_out):
    g = jnp.tanh((accp[0] + accp[1] + hp[...]) * dinv[...] + b[...])
    h_out[...] = jnp.dot(g, wn[...], preferred_element_type=F32) * dinv[...]


def _tmid(accp, hp, dinv, b, wn):
    return pl.pallas_call(
        _tmid_body,
        out_shape=jax.ShapeDtypeStruct((NP, 64), F32),
    )(accp, hp, dinv, b, wn)


def _row_stats(g):
    sq = jnp.sum(g * g, axis=1, keepdims=True)
    nrm = jnp.sqrt(sq)
    ninv = 1.0 / jnp.clip(nrm, 1e-12, None)
    selfdot = sq * ninv * ninv
    return ninv, selfdot


def _pack_xt(g, ninv):
    return jnp.concatenate(
        [g, jnp.ones((NP, 1), F32), ninv, jnp.zeros((NP, W72 - 66), F32)],
        axis=1)


def _t5_body(accp, hp, dinv, b, xt_out, selfdot_out):
    g = jnp.tanh((accp[0] + accp[1] + hp[...]) * dinv[...] + b[...])
    ninv, selfdot = _row_stats(g)
    xt_out[...] = _pack_xt(g, ninv)
    selfdot_out[...] = selfdot


def _t5(accp, hp, dinv, b):
    return pl.pallas_call(
        _t5_body,
        out_shape=[jax.ShapeDtypeStruct((NP, W72), F32),
                   jax.ShapeDtypeStruct((NP, 1), F32)],
    )(accp, hp, dinv, b)


def _t6_body(sden, xt, selfdot, beta, xt_out, selfdot_out):
    S = sden[0, :, 0:64] + sden[1, :, 0:64]
    den = sden[0, :, 64:65] + sden[1, :, 64:65]
    g_prev = xt[:, 0:64]
    exs = jnp.exp(beta[...] * selfdot[...])
    g = jnp.tanh((S + exs * g_prev) / (den + exs + 1e-16))
    ninv, sd2 = _row_stats(g)
    xt_out[...] = _pack_xt(g, ninv)
    selfdot_out[...] = sd2


def _t6(sden, xt, selfdot, beta):
    return pl.pallas_call(
        _t6_body,
        out_shape=[jax.ShapeDtypeStruct((NP, W72), F32),
                   jax.ShapeDtypeStruct((NP, 1), F32)],
    )(sden, xt, selfdot, beta)


def _t7_body(sden, xt, selfdot, beta, batchp, wl, bl, out):
    S = sden[0, :, 0:64] + sden[1, :, 0:64]
    den = sden[0, :, 64:65] + sden[1, :, 64:65]
    g_prev = xt[:, 0:64]
    exs = jnp.exp(beta[...] * selfdot[...])
    g = jnp.tanh((S + exs * g_prev) / (den + exs + 1e-16))
    onehot = (batchp[...] == lax.broadcasted_iota(jnp.int32, (NP, G), 1)).astype(F32)
    ga = jnp.concatenate([g, jnp.ones((NP, 1), F32), jnp.zeros((NP, 15), F32)], axis=1)
    M = lax.dot_general(onehot, ga, (((0,), (0,)), ((), ())),
                        preferred_element_type=F32)
    sums = M[:, 0:64]
    cnt = M[:, 64:65]
    pooled = sums / jnp.clip(cnt, 1.0, None)
    mu = jnp.mean(pooled, axis=-1, keepdims=True)
    dlt = pooled - mu
    var = jnp.mean(dlt * dlt, axis=-1, keepdims=True)
    normed = dlt * lax.rsqrt(var + 1e-5)
    out[...] = jnp.dot(normed, wl[...], preferred_element_type=F32) + bl[...]


def _t7(sden, xt, selfdot, beta, batchp, wlp, bl):
    return pl.pallas_call(
        _t7_body,
        out_shape=jax.ShapeDtypeStruct((G, 8), F32),
    )(sden, xt, selfdot, beta, batchp, wlp, bl)


def kernel(x, edge_index, batch, W1, b1, W2, b2, W3, b3, W4, b4, W5, b5,
           beta1, beta2, Wl, bl):
    src = edge_index[0]
    dst = edge_index[1]
    src2 = jnp.concatenate([src, jnp.zeros((EP - E,), jnp.int32)]).reshape(32, K, C)
    dst2 = jnp.concatenate([dst, jnp.full((EP - E,), TRASH, jnp.int32)]).reshape(32, K, C)
    xp = jnp.pad(x, ((0, NP - N), (0, 0)))
    batchp = jnp.concatenate([batch, jnp.full((NP - N,), -1, jnp.int32)]).reshape(NP, 1)
    ones128 = jnp.ones((C, 16), F32)
    z16 = jnp.zeros((NP, 16), F32)
    z64 = jnp.zeros((NP, 64), F32)
    z72 = jnp.zeros((NP, W72), F32)
    wlp = jnp.pad(Wl, ((0, 0), (0, 7)))

    degp = _sc_deg(dst2, ones128, z16)
    hp, dinv = _t0(degp, xp, W1)
    for b_i, wn in ((b1, W2), (b2, W3), (b3, W4), (b4, W5)):
        acc = _sc_gcn(hp, src2, dst2, z64)
        hp = _tmid(acc, hp, dinv, b_i.reshape(1, 64), wn)
    acc = _sc_gcn(hp, src2, dst2, z64)
    xt, selfdot = _t5(acc, hp, dinv, b5.reshape(1, 64))

    beta1_16 = jnp.broadcast_to(beta1, (16,))
    beta2_16 = jnp.broadcast_to(beta2, (16,))
    sden = _sc_agnn(xt, src2, dst2, beta1_16, z72)
    xt, selfdot = _t6(sden, xt, selfdot, beta1.reshape(1, 1))
    sden = _sc_agnn(xt, src2, dst2, beta2_16, z72)
    out8 = _t7(sden, xt, selfdot, beta2.reshape(1, 1), batchp, wlp,
               bl.reshape(1, 1))
    return out8[:, 0:1]


# speedup vs baseline: 1.1473x; 1.1473x over previous
import functools

import jax
import jax.numpy as jnp
from jax import lax
from jax.experimental import pallas as pl
from jax.experimental.pallas import tpu as pltpu
from jax.experimental.pallas import tpu_sc as plsc

N = 10000
E = 320000
G = 64
NP = 10240
RPT = NP // 16
EP = 327680
C = 128
K = 80
TRASH = 10016
W72 = 72
F32 = jnp.float32

_sc_mesh = functools.partial(
    plsc.VectorSubcoreMesh, core_axis_name="c", subcore_axis_name="s")


def _wid():
    return lax.axis_index("s") * 2 + lax.axis_index("c")


def _make_deg():
    def body(dst3, ones, zeros, out, idx_d, ones_v, acc, semA, semB):
        c = lax.axis_index("c")
        s = lax.axis_index("s")
        w = _wid()
        pltpu.sync_copy(ones, ones_v)
        pltpu.sync_copy(zeros.at[pl.ds(s * RPT, RPT)], acc.at[pl.ds(s * RPT, RPT)])
        pltpu.sync_copy(dst3.at[w], idx_d)
        plsc.subcore_barrier()

        def scat(j, sem):
            pltpu.async_copy(ones_v, acc.at[idx_d.at[j]], sem, add=True)

        def wait_scat(j, sem):
            pltpu.make_async_copy(ones_v, acc.at[idx_d.at[j]], sem).wait()

        scat(0, semA)

        def pair(k, carry):
            j0 = 2 * k + 1
            scat(j0, semB)
            wait_scat(j0 - 1, semA)
            scat(j0 + 1, semA)
            wait_scat(j0, semB)
            return carry

        lax.fori_loop(0, (K - 2) // 2, pair, 0)
        scat(K - 1, semB)
        wait_scat(K - 2, semA)
        wait_scat(K - 1, semB)
        plsc.subcore_barrier()
        pltpu.sync_copy(acc.at[pl.ds(s * RPT, RPT)],
                        out.at[c].at[pl.ds(s * RPT, RPT)])

    return pl.kernel(
        body,
        out_type=jax.ShapeDtypeStruct((2, NP, 16), F32),
        mesh=_sc_mesh(),
        compiler_params=pltpu.CompilerParams(
            use_tc_tiling_on_sc=False, needs_layout_passes=False),
        scratch_types=[
            pltpu.VMEM((K, C), jnp.int32),
            pltpu.VMEM((C, 16), F32),
            pltpu.VMEM_SHARED((NP, 16), F32),
            pltpu.SemaphoreType.DMA,
            pltpu.SemaphoreType.DMA,
        ],
    )


def _make_gcn():
    def body(hp, src3, dst3, zeros, out, idx_s, idx_d, rows0, rows1,
             acc, sem0, sem1, ssc0, ssc1):
        c = lax.axis_index("c")
        s = lax.axis_index("s")
        w = _wid()
        pltpu.sync_copy(zeros.at[pl.ds(s * RPT, RPT)], acc.at[pl.ds(s * RPT, RPT)])
        pltpu.sync_copy(src3.at[w], idx_s.at[pl.ds(0, K)])
        pltpu.sync_copy(dst3.at[w], idx_d)
        for k in range(C // 16):
            idx_s[K, pl.ds(k * 16, 16)] = jnp.zeros((16,), jnp.int32)
        plsc.subcore_barrier()

        def gather(j, buf, sem):
            return pltpu.async_copy(hp.at[idx_s.at[j]], buf, sem)

        def wait_gather(j, buf, sem):
            pltpu.make_async_copy(hp.at[idx_s.at[j]], buf, sem).wait()

        def scat(j, buf, sem):
            pltpu.async_copy(buf, acc.at[idx_d.at[j]], sem, add=True)

        def wait_scat(j, buf, sem):
            pltpu.make_async_copy(buf, acc.at[idx_d.at[j]], sem).wait()

        gather(0, rows0, ssc0)
        wait_gather(0, rows0, ssc0)
        scat(0, rows0, sem0)
        gather(1, rows1, ssc1)

        def pair(k, carry):
            j0 = 2 * k + 1
            wait_gather(j0, rows1, ssc1)
            scat(j0, rows1, sem1)
            wait_scat(j0 - 1, rows0, sem0)
            gather(j0 + 1, rows0, ssc0)
            wait_gather(j0 + 1, rows0, ssc0)
            scat(j0 + 1, rows0, sem0)
            wait_scat(j0, rows1, sem1)
            gather(j0 + 2, rows1, ssc1)
            return carry

        lax.fori_loop(0, (K - 2) // 2, pair, 0)
        wait_gather(K - 1, rows1, ssc1)
        scat(K - 1, rows1, sem1)
        wait_scat(K - 2, rows0, sem0)
        wait_scat(K - 1, rows1, sem1)

        plsc.subcore_barrier()
        pltpu.sync_copy(acc.at[pl.ds(s * RPT, RPT)],
                        out.at[c].at[pl.ds(s * RPT, RPT)])

    return pl.kernel(
        body,
        out_type=jax.ShapeDtypeStruct((2, NP, 64), F32),
        mesh=_sc_mesh(),
        compiler_params=pltpu.CompilerParams(
            use_tc_tiling_on_sc=False, needs_layout_passes=False),
        scratch_types=[
            pltpu.VMEM((K + 1, C), jnp.int32),
            pltpu.VMEM((K, C), jnp.int32),
            pltpu.VMEM((C, 64), F32),
            pltpu.VMEM((C, 64), F32),
            pltpu.VMEM_SHARED((NP, 64), F32),
            pltpu.SemaphoreType.DMA,
            pltpu.SemaphoreType.DMA,
            pltpu.SemaphoreType.DMA,
            pltpu.SemaphoreType.DMA,
        ],
    )


def _make_agnn():
    def compute_chunk(rows_s, rows_d, bvec):
        c65 = jnp.full((16,), 65, jnp.int32)

        def grp(gi, carry):
            cvec = lax.iota(jnp.int32, 16) + gi * 16

            def fq(q, a):
                for k in range(4):
                    fv = lax.broadcast(q * 4 + k, (16,))
                    vs = plsc.load_gather(rows_s, [cvec, fv])
                    vd = plsc.load_gather(rows_d, [cvec, fv])
                    a = a + vs * vd
                return a

            dot = lax.fori_loop(0, 16, fq, jnp.zeros((16,), F32))
            ns = plsc.load_gather(rows_s, [cvec, c65])
            nd = plsc.load_gather(rows_d, [cvec, c65])
            ex = jnp.exp(bvec * dot * ns * nd)

            def fscale(f, carry2):
                fv = lax.broadcast(f, (16,))
                col = plsc.load_gather(rows_s, [cvec, fv])
                plsc.store_scatter(rows_s, [cvec, fv], col * ex)
                return carry2

            lax.fori_loop(0, 65, fscale, 0)
            return carry

        lax.fori_loop(0, C // 16, grp, 0)

    def body(xt, src3, dst3, beta16, zeros, out, idx_s, idx_d,
             rs0, rs1, rd0, rd1, beta_v, acc, ss0, ss1, sd0, sd1, sc0, sc1):
        c = lax.axis_index("c")
        s = lax.axis_index("s")
        w = _wid()
        pltpu.sync_copy(beta16, beta_v)
        pltpu.sync_copy(zeros.at[pl.ds(s * RPT, RPT)], acc.at[pl.ds(s * RPT, RPT)])
        pltpu.sync_copy(src3.at[w], idx_s.at[pl.ds(0, K)])
        pltpu.sync_copy(dst3.at[w], idx_d.at[pl.ds(0, K)])
        for k in range(C // 16):
            idx_s[K, pl.ds(k * 16, 16)] = jnp.zeros((16,), jnp.int32)
            idx_d[K, pl.ds(k * 16, 16)] = jnp.zeros((16,), jnp.int32)
        plsc.subcore_barrier()
        bvec = beta_v[...]

        def gathers(j, bs, bd, sems, semd):
            pltpu.async_copy(xt.at[idx_s.at[j]], bs, sems)
            pltpu.async_copy(xt.at[idx_d.at[j]], bd, semd)

        def waits(j, bs, bd, sems, semd):
            pltpu.make_async_copy(xt.at[idx_s.at[j]], bs, sems).wait()
            pltpu.make_async_copy(xt.at[idx_d.at[j]], bd, semd).wait()

        def scat(j, buf, sem):
            pltpu.async_copy(buf, acc.at[idx_d.at[j]], sem, add=True)

        def wait_scat(j, buf, sem):
            pltpu.make_async_copy(buf, acc.at[idx_d.at[j]], sem).wait()

        gathers(0, rs0, rd0, ss0, sd0)
        waits(0, rs0, rd0, ss0, sd0)
        gathers(1, rs1, rd1, ss1, sd1)
        compute_chunk(rs0, rd0, bvec)
        scat(0, rs0, sc0)

        def pair(k, carry):
            j0 = 2 * k + 1
            waits(j0, rs1, rd1, ss1, sd1)
            wait_scat(j0 - 1, rs0, sc0)
            gathers(j0 + 1, rs0, rd0, ss0, sd0)
            compute_chunk(rs1, rd1, bvec)
            scat(j0, rs1, sc1)
            waits(j0 + 1, rs0, rd0, ss0, sd0)
            wait_scat(j0, rs1, sc1)
            gathers(j0 + 2, rs1, rd1, ss1, sd1)
            compute_chunk(rs0, rd0, bvec)
            scat(j0 + 1, rs0, sc0)
            return carry

        lax.fori_loop(0, (K - 2) // 2, pair, 0)
        waits(K - 1, rs1, rd1, ss1, sd1)
        wait_scat(K - 2, rs0, sc0)
        compute_chunk(rs1, rd1, bvec)
        scat(K - 1, rs1, sc1)
        wait_scat(K - 1, rs1, sc1)

        plsc.subcore_barrier()
        pltpu.sync_copy(acc.at[pl.ds(s * RPT, RPT)],
                        out.at[c].at[pl.ds(s * RPT, RPT)])

    return pl.kernel(
        body,
        out_type=jax.ShapeDtypeStruct((2, NP, W72), F32),
        mesh=_sc_mesh(),
        compiler_params=pltpu.CompilerParams(
            use_tc_tiling_on_sc=False, needs_layout_passes=False),
        scratch_types=[
            pltpu.VMEM((K + 1, C), jnp.int32),
            pltpu.VMEM((K + 1, C), jnp.int32),
            pltpu.VMEM((C, W72), F32),
            pltpu.VMEM((C, W72), F32),
            pltpu.VMEM((C, W72), F32),
            pltpu.VMEM((C, W72), F32),
            pltpu.VMEM((16,), F32),
            pltpu.VMEM_SHARED((NP, W72), F32),
            pltpu.SemaphoreType.DMA,
            pltpu.SemaphoreType.DMA,
            pltpu.SemaphoreType.DMA,
            pltpu.SemaphoreType.DMA,
            pltpu.SemaphoreType.DMA,
            pltpu.SemaphoreType.DMA,
        ],
    )


_sc_deg = _make_deg()
_sc_gcn = _make_gcn()
_sc_agnn = _make_agnn()


def _t0_body(degp, xp, w1, h_out, dinv_out):
    deg = degp[0, :, 0:1] + degp[1, :, 0:1] + 1.0
    dinv = lax.rsqrt(deg)
    dinv_out[...] = dinv
    h_out[...] = jnp.dot(xp[...], w1[...], preferred_element_type=F32) * dinv


def _t0(degp, xp, w1):
    return pl.pallas_call(
        _t0_body,
        out_shape=[jax.ShapeDtypeStruct((NP, 64), F32),
                   jax.ShapeDtypeStruct((NP, 1), F32)],
    )(degp, xp, w1)


def _tmid_body(accp, hp, dinv, b, wn, h_out):
    g = jnp.tanh((accp[0] + accp[1] + hp[...]) * dinv[...] + b[...])
    h_out[...] = jnp.dot(g, wn[...], preferred_element_type=F32) * dinv[...]


def _tmid(accp, hp, dinv, b, wn):
    return pl.pallas_call(
        _tmid_body,
        out_shape=jax.ShapeDtypeStruct((NP, 64), F32),
    )(accp, hp, dinv, b, wn)


def _row_stats(g):
    sq = jnp.sum(g * g, axis=1, keepdims=True)
    nrm = jnp.sqrt(sq)
    ninv = 1.0 / jnp.clip(nrm, 1e-12, None)
    selfdot = sq * ninv * ninv
    return ninv, selfdot


def _pack_xt(g, ninv):
    return jnp.concatenate(
        [g, jnp.ones((NP, 1), F32), ninv, jnp.zeros((NP, W72 - 66), F32)],
        axis=1)


def _t5_body(accp, hp, dinv, b, xt_out, selfdot_out):
    g = jnp.tanh((accp[0] + accp[1] + hp[...]) * dinv[...] + b[...])
    ninv, selfdot = _row_stats(g)
    xt_out[...] = _pack_xt(g, ninv)
    selfdot_out[...] = selfdot


def _t5(accp, hp, dinv, b):
    return pl.pallas_call(
        _t5_body,
        out_shape=[jax.ShapeDtypeStruct((NP, W72), F32),
                   jax.ShapeDtypeStruct((NP, 1), F32)],
    )(accp, hp, dinv, b)


def _t6_body(sden, xt, selfdot, beta, xt_out, selfdot_out):
    S = sden[0, :, 0:64] + sden[1, :, 0:64]
    den = sden[0, :, 64:65] + sden[1, :, 64:65]
    g_prev = xt[:, 0:64]
    exs = jnp.exp(beta[...] * selfdot[...])
    g = jnp.tanh((S + exs * g_prev) / (den + exs + 1e-16))
    ninv, sd2 = _row_stats(g)
    xt_out[...] = _pack_xt(g, ninv)
    selfdot_out[...] = sd2


def _t6(sden, xt, selfdot, beta):
    return pl.pallas_call(
        _t6_body,
        out_shape=[jax.ShapeDtypeStruct((NP, W72), F32),
                   jax.ShapeDtypeStruct((NP, 1), F32)],
    )(sden, xt, selfdot, beta)


def _t7_body(sden, xt, selfdot, beta, batchp, wl, bl, out):
    S = sden[0, :, 0:64] + sden[1, :, 0:64]
    den = sden[0, :, 64:65] + sden[1, :, 64:65]
    g_prev = xt[:, 0:64]
    exs = jnp.exp(beta[...] * selfdot[...])
    g = jnp.tanh((S + exs * g_prev) / (den + exs + 1e-16))
    onehot = (batchp[...] == lax.broadcasted_iota(jnp.int32, (NP, G), 1)).astype(F32)
    ga = jnp.concatenate([g, jnp.ones((NP, 1), F32), jnp.zeros((NP, 15), F32)], axis=1)
    M = lax.dot_general(onehot, ga, (((0,), (0,)), ((), ())),
                        preferred_element_type=F32)
    sums = M[:, 0:64]
    cnt = M[:, 64:65]
    pooled = sums / jnp.clip(cnt, 1.0, None)
    mu = jnp.mean(pooled, axis=-1, keepdims=True)
    dlt = pooled - mu
    var = jnp.mean(dlt * dlt, axis=-1, keepdims=True)
    normed = dlt * lax.rsqrt(var + 1e-5)
    out[...] = jnp.dot(normed, wl[...], preferred_element_type=F32) + bl[...]


def _t7(sden, xt, selfdot, beta, batchp, wlp, bl):
    return pl.pallas_call(
        _t7_body,
        out_shape=jax.ShapeDtypeStruct((G, 8), F32),
    )(sden, xt, selfdot, beta, batchp, wlp, bl)


def kernel(x, edge_index, batch, W1, b1, W2, b2, W3, b3, W4, b4, W5, b5,
           beta1, beta2, Wl, bl):
    src = edge_index[0]
    dst = edge_index[1]
    src2 = jnp.concatenate([src, jnp.zeros((EP - E,), jnp.int32)]).reshape(32, K, C)
    dst2 = jnp.concatenate([dst, jnp.full((EP - E,), TRASH, jnp.int32)]).reshape(32, K, C)
    xp = jnp.pad(x, ((0, NP - N), (0, 0)))
    batchp = jnp.concatenate([batch, jnp.full((NP - N,), -1, jnp.int32)]).reshape(NP, 1)
    ones128 = jnp.ones((C, 16), F32)
    z16 = jnp.zeros((NP, 16), F32)
    z64 = jnp.zeros((NP, 64), F32)
    z72 = jnp.zeros((NP, W72), F32)
    wlp = jnp.pad(Wl, ((0, 0), (0, 7)))

    degp = _sc_deg(dst2, ones128, z16)
    hp, dinv = _t0(degp, xp, W1)
    for b_i, wn in ((b1, W2), (b2, W3), (b3, W4), (b4, W5)):
        acc = _sc_gcn(hp, src2, dst2, z64)
        hp = _tmid(acc, hp, dinv, b_i.reshape(1, 64), wn)
    acc = _sc_gcn(hp, src2, dst2, z64)
    xt, selfdot = _t5(acc, hp, dinv, b5.reshape(1, 64))

    beta1_16 = jnp.broadcast_to(beta1, (16,))
    beta2_16 = jnp.broadcast_to(beta2, (16,))
    sden = _sc_agnn(xt, src2, dst2, beta1_16, z72)
    xt, selfdot = _t6(sden, xt, selfdot, beta1.reshape(1, 1))
    sden = _sc_agnn(xt, src2, dst2, beta2_16, z72)
    out8 = _t7(sden, xt, selfdot, beta2.reshape(1, 1), batchp, wlp,
               bl.reshape(1, 1))
    return out8[:, 0:1]
